# baseline (speedup 1.0000x reference)
import functools

import jax
import jax.numpy as jnp
from jax import lax
from jax.experimental import pallas as pl
from jax.experimental.pallas import tpu as pltpu
from jax.experimental.pallas import tpu_sc as plsc

_VOCAB = 1000
_D = 128
_BATCH = 16384

_NC = 2
_NS = 16
_NW = _NC * _NS
_CH = 128
_ORPW = _BATCH // _NW
_NCH = _ORPW // _CH


def _proj_body(e_ref, w_ref, b_ref, o_ref):
    o_ref[...] = lax.dot_general(
        e_ref[...], w_ref[...],
        dimension_numbers=(((1,), (1,)), ((), ())),
        preferred_element_type=jnp.float32,
    ) + b_ref[...]


def _project_table(emb_weight, lin_w, lin_b):
    return pl.pallas_call(
        _proj_body,
        out_shape=jax.ShapeDtypeStruct((_VOCAB, _D), jnp.float32),
    )(emb_weight, lin_w, lin_b.reshape(1, _D))


_mesh = plsc.VectorSubcoreMesh(core_axis_name="c", subcore_axis_name="s")

_NB = 6
_AH = 4


@functools.partial(
    pl.kernel,
    out_type=jax.ShapeDtypeStruct((_BATCH, 2 * _D), jnp.float32),
    mesh=_mesh,
    scratch_types=[
        pltpu.VMEM((2 * _NCH, _CH), jnp.int32),
        pltpu.VMEM((_NB, _CH, _D), jnp.float32),
    ] + [pltpu.SemaphoreType.DMA] * (2 * _NB),
)
def _gather_rows(idx_hbm, table_hbm, out_hbm, idx_v, rows_v, *sems):
    gsems, wsems = sems[:_NB], sems[_NB:]
    wid = lax.axis_index("s") * _NC + lax.axis_index("c")
    base = wid * _ORPW
    nch = 2 * _NCH
    pltpu.sync_copy(idx_hbm.at[pl.ds(wid * nch, nch)], idx_v)

    def out_slice(j):
        return out_hbm.at[pl.ds(base + (j // 2) * _CH, _CH),
                          pl.ds((j % 2) * _D, _D)]

    def gather(j):
        return pltpu.async_copy(
            table_hbm.at[idx_v.at[j]], rows_v.at[j % _NB], gsems[j % _NB])

    g_cps = [None] * nch
    w_cps = [None] * nch
    for j in range(_AH):
        g_cps[j] = gather(j)
    for j in range(nch):
        g_cps[j].wait()
        w_cps[j] = pltpu.async_copy(rows_v.at[j % _NB], out_slice(j),
                                    wsems[j % _NB])
        k = j + _AH
        if k < nch:
            if k >= _NB:
                w_cps[k - _NB].wait()
            g_cps[k] = gather(k)
    for j in range(nch - _NB, nch):
        if j >= 0 and w_cps[j] is not None:
            w_cps[j].wait()


def kernel(bandwidth, emb_weight, lin_w, lin_b):
    table = _project_table(emb_weight, lin_w, lin_b)
    idx = (bandwidth.astype(jnp.int32)
           .reshape(_BATCH // _CH, _CH, 2)
           .transpose(0, 2, 1)
           .reshape(2 * _BATCH // _CH, _CH))
    return _gather_rows(idx, table)


# speedup vs baseline: 1.3536x; 1.3536x over previous
import functools

import jax
import jax.numpy as jnp
from jax import lax
from jax.experimental import pallas as pl
from jax.experimental.pallas import tpu as pltpu
from jax.experimental.pallas import tpu_sc as plsc

_VOCAB = 1000
_D = 128
_BATCH = 16384

_NC = 2
_NS = 16
_NW = _NC * _NS
_CH = 128
_ORPW = _BATCH // _NW
_NCH = _ORPW // _CH


def _proj_body(e_ref, w_ref, b_ref, o_ref):
    o_ref[...] = lax.dot_general(
        e_ref[...], w_ref[...],
        dimension_numbers=(((1,), (1,)), ((), ())),
        preferred_element_type=jnp.float32,
    ) + b_ref[...]


def _project_table(emb_weight, lin_w, lin_b):
    return pl.pallas_call(
        _proj_body,
        out_shape=jax.ShapeDtypeStruct((_VOCAB, _D), jnp.float32),
    )(emb_weight, lin_w, lin_b.reshape(1, _D))


_mesh = plsc.VectorSubcoreMesh(core_axis_name="c", subcore_axis_name="s")

_NB = 6
_AH = 4


@functools.partial(
    pl.kernel,
    out_type=jax.ShapeDtypeStruct((_BATCH, 2 * _D), jnp.float32),
    mesh=_mesh,
    scratch_types=[
        pltpu.VMEM((2 * _NCH, _CH), jnp.int32),
        pltpu.VMEM((_NB, _CH, _D), jnp.float32),
        pltpu.VMEM_SHARED((_VOCAB, _D), jnp.float32),
    ] + [pltpu.SemaphoreType.DMA] * (2 * _NB),
)
def _gather_rows(idx_hbm, table_hbm, out_hbm, idx_v, rows_v, table_sh,
                 *sems):
    gsems, wsems = sems[:_NB], sems[_NB:]
    wid = lax.axis_index("s") * _NC + lax.axis_index("c")
    base = wid * _ORPW
    nch = 2 * _NCH
    @pl.when(lax.axis_index("s") == 0)
    def _stage_table():
        pltpu.sync_copy(table_hbm, table_sh)

    pltpu.sync_copy(idx_hbm.at[pl.ds(wid * nch, nch)], idx_v)
    plsc.subcore_barrier()

    def out_slice(j):
        return out_hbm.at[pl.ds(base + (j // 2) * _CH, _CH),
                          pl.ds((j % 2) * _D, _D)]

    def gather(j):
        return pltpu.async_copy(
            table_sh.at[idx_v.at[j]], rows_v.at[j % _NB], gsems[j % _NB])

    g_cps = [None] * nch
    w_cps = [None] * nch
    for j in range(_AH):
        g_cps[j] = gather(j)
    for j in range(nch):
        g_cps[j].wait()
        w_cps[j] = pltpu.async_copy(rows_v.at[j % _NB], out_slice(j),
                                    wsems[j % _NB])
        k = j + _AH
        if k < nch:
            if k >= _NB:
                w_cps[k - _NB].wait()
            g_cps[k] = gather(k)
    for j in range(nch - _NB, nch):
        if j >= 0 and w_cps[j] is not None:
            w_cps[j].wait()


def kernel(bandwidth, emb_weight, lin_w, lin_b):
    table = _project_table(emb_weight, lin_w, lin_b)
    idx = (bandwidth.astype(jnp.int32)
           .reshape(_BATCH // _CH, _CH, 2)
           .transpose(0, 2, 1)
           .reshape(2 * _BATCH // _CH, _CH))
    return _gather_rows(idx, table)


# speedup vs baseline: 1.3897x; 1.0266x over previous
import functools

import jax
import jax.numpy as jnp
from jax import lax
from jax.experimental import pallas as pl
from jax.experimental.pallas import tpu as pltpu
from jax.experimental.pallas import tpu_sc as plsc

_VOCAB = 1000
_D = 128
_BATCH = 16384

_NC = 2
_NS = 16
_NW = _NC * _NS
_CH = 128
_ORPW = _BATCH // _NW
_NCH = _ORPW // _CH


def _proj_body(e_ref, w_ref, b_ref, o_ref):
    o_ref[...] = lax.dot_general(
        e_ref[...], w_ref[...],
        dimension_numbers=(((1,), (1,)), ((), ())),
        preferred_element_type=jnp.float32,
    ) + b_ref[...]


def _project_table(emb_weight, lin_w, lin_b):
    return pl.pallas_call(
        _proj_body,
        out_shape=jax.ShapeDtypeStruct((_VOCAB, _D), jnp.float32),
    )(emb_weight, lin_w, lin_b.reshape(1, _D))


_mesh = plsc.VectorSubcoreMesh(core_axis_name="c", subcore_axis_name="s")

_NB = 6
_AH = 4


@functools.partial(
    pl.kernel,
    out_type=jax.ShapeDtypeStruct((_BATCH, 2 * _D), jnp.float32),
    mesh=_mesh,
    scratch_types=[
        pltpu.VMEM((2 * _NCH, _CH), jnp.int32),
        pltpu.VMEM((_NB, _CH, _D), jnp.float32),
        pltpu.VMEM_SHARED((_VOCAB, _D), jnp.float32),
    ] + [pltpu.SemaphoreType.DMA] * (2 * _NB),
)
def _gather_rows(idx_hbm, table_hbm, out_hbm, idx_v, rows_v, table_sh,
                 *sems):
    gsems, wsems = sems[:_NB], sems[_NB:]
    wid = lax.axis_index("s") * _NC + lax.axis_index("c")
    base = wid * _ORPW
    nch = 2 * _NCH
    sid = lax.axis_index("s")
    rpt = 64
    off = pl.multiple_of(jnp.minimum(sid * rpt, _VOCAB - rpt), 8)
    tcp = pltpu.async_copy(
        table_hbm.at[pl.ds(off, rpt)], table_sh.at[pl.ds(off, rpt)],
        gsems[0])

    pltpu.sync_copy(idx_hbm.at[pl.ds(wid * nch, nch)], idx_v)
    tcp.wait()
    plsc.subcore_barrier()

    def out_slice(j):
        return out_hbm.at[pl.ds(base + (j // 2) * _CH, _CH),
                          pl.ds((j % 2) * _D, _D)]

    def gather(j):
        return pltpu.async_copy(
            table_sh.at[idx_v.at[j]], rows_v.at[j % _NB], gsems[j % _NB])

    g_cps = [None] * nch
    w_cps = [None] * nch
    for j in range(_AH):
        g_cps[j] = gather(j)
    for j in range(nch):
        g_cps[j].wait()
        w_cps[j] = pltpu.async_copy(rows_v.at[j % _NB], out_slice(j),
                                    wsems[j % _NB])
        k = j + _AH
        if k < nch:
            if k >= _NB:
                w_cps[k - _NB].wait()
            g_cps[k] = gather(k)
    for j in range(nch - _NB, nch):
        if j >= 0 and w_cps[j] is not None:
            w_cps[j].wait()


def kernel(bandwidth, emb_weight, lin_w, lin_b):
    table = _project_table(emb_weight, lin_w, lin_b)
    idx = (bandwidth.astype(jnp.int32)
           .reshape(_BATCH // _CH, _CH, 2)
           .transpose(0, 2, 1)
           .reshape(2 * _BATCH // _CH, _CH))
    return _gather_rows(idx, table)
